# baseline (speedup 1.0000x reference)
import jax
import jax.numpy as jnp
from jax.experimental import pallas as pl
from jax.experimental.pallas import tpu as pltpu
from jax.experimental.pallas import tpu_sc as plsc

WINDOW = 64
GATHERS_PER_STEP = 4
BLOCK = WINDOW * GATHERS_PER_STEP


def kernel(input_ids, embedding):
    batch, seq = input_ids.shape
    vocab, dim = embedding.shape
    num_idx = batch * seq
    assert num_idx % BLOCK == 0

    idx = input_ids.reshape(1, num_idx).astype(jnp.int32)
    mesh = plsc.VectorSubcoreMesh(core_axis_name="core", subcore_axis_name="subcore")

    @jax.jit
    def gather(table, idx):
        @pl.kernel(
            out_type=jax.ShapeDtypeStruct((num_idx, dim), table.dtype),
            mesh=mesh,
            scratch_types=[pltpu.SemaphoreType.DMA],
        )
        def gather_kernel(table_hbm, idx_hbm, out_hbm, sem):
            def body(i_vmem, o_vmem):
                copies = [
                    pltpu.async_copy(
                        table_hbm.at[i_vmem.at[0, pl.ds(g * WINDOW, WINDOW)]],
                        o_vmem.at[pl.ds(g * WINDOW, WINDOW)],
                        sem,
                    )
                    for g in range(GATHERS_PER_STEP)
                ]
                for c in copies:
                    c.wait()

            pltpu.emit_pipeline(
                body,
                grid=(num_idx // BLOCK,),
                in_specs=[pl.BlockSpec((1, BLOCK), lambda i: (0, i))],
                out_specs=[pl.BlockSpec((BLOCK, dim), lambda i: (i, 0))],
                core_axis_name=("core", "subcore"),
                dimension_semantics=(pltpu.PARALLEL,),
            )(idx_hbm, out_hbm)

        return gather_kernel(table, idx)

    out = gather(embedding, idx)
    return out.reshape(batch, seq, dim)


# speedup vs baseline: 1.0174x; 1.0174x over previous
import jax
import jax.numpy as jnp
from jax.experimental import pallas as pl
from jax.experimental.pallas import tpu as pltpu
from jax.experimental.pallas import tpu_sc as plsc
from jax import lax

NUM_CORES = 2
NUM_SUBCORES = 16
NUM_WORKERS = NUM_CORES * NUM_SUBCORES

CHUNK = 128
NBUF = 5


def kernel(input_ids, embedding):
    batch, seq = input_ids.shape
    vocab, dim = embedding.shape
    num_idx = batch * seq
    per_worker = num_idx // NUM_WORKERS
    n_chunks = per_worker // CHUNK
    n_groups = n_chunks // NBUF
    assert per_worker * NUM_WORKERS == num_idx
    assert n_groups * NBUF == n_chunks

    idx = input_ids.reshape(num_idx).astype(jnp.int32)
    mesh = plsc.VectorSubcoreMesh(core_axis_name="core", subcore_axis_name="subcore")

    @jax.jit
    def gather(table, idx):
        @pl.kernel(
            out_type=jax.ShapeDtypeStruct((num_idx, dim), table.dtype),
            mesh=mesh,
            scratch_types=[
                pltpu.VMEM((per_worker,), jnp.int32),
                pltpu.VMEM((NBUF, CHUNK, dim), table.dtype),
                pltpu.SemaphoreType.DMA,
                pltpu.SemaphoreType.DMA((NBUF,)),
                pltpu.SemaphoreType.DMA((NBUF,)),
            ],
        )
        def gather_kernel(table_hbm, idx_hbm, out_hbm, idx_v, rows, isem, gsem, ssem):
            wid = lax.axis_index("subcore") * NUM_CORES + lax.axis_index("core")
            base = wid * per_worker
            pltpu.async_copy(idx_hbm.at[pl.ds(base, per_worker)], idx_v, isem).wait()

            def fire_gather(c, b):
                return pltpu.async_copy(
                    table_hbm.at[idx_v.at[pl.ds(c * CHUNK, CHUNK)]],
                    rows.at[b],
                    gsem.at[b],
                )

            def fire_store(c, b):
                pltpu.async_copy(
                    rows.at[b],
                    out_hbm.at[pl.ds(base + c * CHUNK, CHUNK)],
                    ssem.at[b],
                )

            def wait_gather(b):
                pltpu.make_async_copy(
                    table_hbm.at[pl.ds(0, CHUNK)], rows.at[b], gsem.at[b]
                ).wait()

            def wait_store(b):
                pltpu.make_async_copy(
                    out_hbm.at[pl.ds(0, CHUNK)], rows.at[b], ssem.at[b]
                ).wait()

            for b in range(NBUF):
                fire_gather(b, b)
            for b in range(NBUF):
                wait_gather(b)
                fire_store(b, b)

            @pl.loop(1, n_groups)
            def group(o):
                c0 = o * NBUF
                for b in range(NBUF):
                    wait_store(b)
                    fire_gather(c0 + b, b)
                for b in range(NBUF):
                    wait_gather(b)
                    fire_store(c0 + b, b)

            for b in range(NBUF):
                pltpu.make_async_copy(
                    out_hbm.at[pl.ds(0, CHUNK)], rows.at[b], ssem.at[b]
                ).wait()

        return gather_kernel(table, idx)

    out = gather(embedding, idx)
    return out.reshape(batch, seq, dim)


# speedup vs baseline: 1.0262x; 1.0086x over previous
import jax
import jax.numpy as jnp
from jax.experimental import pallas as pl
from jax.experimental.pallas import tpu as pltpu
from jax.experimental.pallas import tpu_sc as plsc
from jax import lax

NUM_CORES = 2
NUM_SUBCORES = 16
NUM_WORKERS = NUM_CORES * NUM_SUBCORES

CHUNK = 64
NBUF = 10


def kernel(input_ids, embedding):
    batch, seq = input_ids.shape
    vocab, dim = embedding.shape
    num_idx = batch * seq
    per_worker = num_idx // NUM_WORKERS
    n_chunks = per_worker // CHUNK
    n_groups = n_chunks // NBUF
    assert per_worker * NUM_WORKERS == num_idx
    assert n_groups * NBUF == n_chunks

    idx = input_ids.reshape(num_idx).astype(jnp.int32)
    mesh = plsc.VectorSubcoreMesh(core_axis_name="core", subcore_axis_name="subcore")

    @jax.jit
    def gather(table, idx):
        @pl.kernel(
            out_type=jax.ShapeDtypeStruct((num_idx, dim), table.dtype),
            mesh=mesh,
            scratch_types=[
                pltpu.VMEM((per_worker,), jnp.int32),
                pltpu.VMEM((NBUF, CHUNK, dim), table.dtype),
                pltpu.SemaphoreType.DMA,
                pltpu.SemaphoreType.DMA((NBUF,)),
                pltpu.SemaphoreType.DMA((NBUF,)),
            ],
        )
        def gather_kernel(table_hbm, idx_hbm, out_hbm, idx_v, rows, isem, gsem, ssem):
            wid = lax.axis_index("subcore") * NUM_CORES + lax.axis_index("core")
            base = wid * per_worker
            pltpu.async_copy(idx_hbm.at[pl.ds(base, per_worker)], idx_v, isem).wait()

            def fire_gather(c, b):
                return pltpu.async_copy(
                    table_hbm.at[idx_v.at[pl.ds(c * CHUNK, CHUNK)]],
                    rows.at[b],
                    gsem.at[b],
                )

            def fire_store(c, b):
                pltpu.async_copy(
                    rows.at[b],
                    out_hbm.at[pl.ds(base + c * CHUNK, CHUNK)],
                    ssem.at[b],
                )

            def wait_gather(b):
                pltpu.make_async_copy(
                    table_hbm.at[pl.ds(0, CHUNK)], rows.at[b], gsem.at[b]
                ).wait()

            def wait_store(b):
                pltpu.make_async_copy(
                    out_hbm.at[pl.ds(0, CHUNK)], rows.at[b], ssem.at[b]
                ).wait()

            for b in range(NBUF):
                fire_gather(b, b)
            for b in range(NBUF):
                wait_gather(b)
                fire_store(b, b)

            @pl.loop(1, n_groups)
            def group(o):
                c0 = o * NBUF
                for b in range(NBUF):
                    wait_store(b)
                    fire_gather(c0 + b, b)
                for b in range(NBUF):
                    wait_gather(b)
                    fire_store(c0 + b, b)

            for b in range(NBUF):
                pltpu.make_async_copy(
                    out_hbm.at[pl.ds(0, CHUNK)], rows.at[b], ssem.at[b]
                ).wait()

        return gather_kernel(table, idx)

    out = gather(embedding, idx)
    return out.reshape(batch, seq, dim)
